# baseline (speedup 1.0000x reference)
import jax
import jax.numpy as jnp
from jax import lax
from jax.experimental import pallas as pl
from jax.experimental.pallas import tpu as pltpu
from jax.experimental.pallas import tpu_sc as plsc

BATCH = 16384
DIM = 128
LANES = 16
N_WORKERS = 32
ROWS_PER_W = BATCH // N_WORKERS
CHUNK = 128
N_CHUNKS = ROWS_PER_W // CHUNK
GSLICE = 128


def _sc_body(z_hbm, j_hbm, ep_hbm, out_hbm,
             idx_all, idxp1_all, lo_all, hi_all,
             z_b0, z_b1, z_b2, z_b3,
             sem_z0, sem_z1, sem_z2, sem_z3,
             sem_o0, sem_o1, sem_o2, sem_o3, sem_g):
    wid = lax.axis_index("s") * 2 + lax.axis_index("c")
    base = wid * ROWS_PER_W
    z_b = [z_b0, z_b1, z_b2, z_b3]
    sem_z = [sem_z0, sem_z1, sem_z2, sem_z3]
    sem_o = [sem_o0, sem_o1, sem_o2, sem_o3]

    fills = [
        pltpu.async_copy(
            z_hbm.at[pl.ds(base + k * CHUNK, CHUNK), :], z_b[k], sem_z[k])
        for k in range(N_CHUNKS)
    ]
    pltpu.sync_copy(j_hbm.at[pl.ds(base, ROWS_PER_W)], idx_all)
    for v in range(ROWS_PER_W // LANES):
        s = pl.ds(v * LANES, LANES)
        idxp1_all[s] = idx_all[s] + 1
    gathers = []
    for g in range(ROWS_PER_W // GSLICE):
        s = pl.ds(g * GSLICE, GSLICE)
        gathers.append(
            pltpu.async_copy(ep_hbm.at[idx_all.at[s]], lo_all.at[s], sem_g))
        gathers.append(
            pltpu.async_copy(ep_hbm.at[idxp1_all.at[s]], hi_all.at[s], sem_g))
    for g in gathers:
        g.wait()

    drains = []
    for k in range(N_CHUNKS):
        off = k * CHUNK
        fills[k].wait()
        zb = z_b[k]

        def grp_body(gi, _):
            o = off + gi * LANES
            lo_vec = lo_all[pl.ds(o, LANES)]
            hi_vec = hi_all[pl.ds(o, LANES)]
            sc_vec = hi_vec - lo_vec
            for r in range(LANES):
                lo_s = lo_vec[r]
                sc_s = sc_vec[r]
                row = gi * LANES + r
                for v in range(DIM // LANES):
                    s = pl.ds(v * LANES, LANES)
                    zb[row, s] = zb[row, s] * sc_s + lo_s
            return 0

        lax.fori_loop(0, CHUNK // LANES, grp_body, 0)
        drains.append(pltpu.async_copy(
            zb, out_hbm.at[pl.ds(base + off, CHUNK), :], sem_o[k]))
    for d in drains:
        d.wait()


@jax.jit
def _shift_mapper_sc(z, j_flat, endpoints):
    mesh = plsc.VectorSubcoreMesh(core_axis_name="c", subcore_axis_name="s")
    kfn = pl.kernel(
        _sc_body,
        mesh=mesh,
        out_type=jax.ShapeDtypeStruct((BATCH, DIM), jnp.float32),
        scratch_types=[
            pltpu.VMEM((ROWS_PER_W,), jnp.int32),
            pltpu.VMEM((ROWS_PER_W,), jnp.int32),
            pltpu.VMEM((ROWS_PER_W,), jnp.float32),
            pltpu.VMEM((ROWS_PER_W,), jnp.float32),
            pltpu.VMEM((CHUNK, DIM), jnp.float32),
            pltpu.VMEM((CHUNK, DIM), jnp.float32),
            pltpu.VMEM((CHUNK, DIM), jnp.float32),
            pltpu.VMEM((CHUNK, DIM), jnp.float32),
            pltpu.SemaphoreType.DMA,
            pltpu.SemaphoreType.DMA,
            pltpu.SemaphoreType.DMA,
            pltpu.SemaphoreType.DMA,
            pltpu.SemaphoreType.DMA,
            pltpu.SemaphoreType.DMA,
            pltpu.SemaphoreType.DMA,
            pltpu.SemaphoreType.DMA,
            pltpu.SemaphoreType.DMA,
        ],
        compiler_params=pltpu.CompilerParams(needs_layout_passes=False),
    )
    return kfn(z, j_flat, endpoints)


def kernel(z, j, endpoints):
    j_flat = j.reshape(-1).astype(jnp.int32)
    return _shift_mapper_sc(z, j_flat, endpoints)


# speedup vs baseline: 1.0543x; 1.0543x over previous
import jax
import jax.numpy as jnp
from jax import lax
from jax.experimental import pallas as pl
from jax.experimental.pallas import tpu as pltpu
from jax.experimental.pallas import tpu_sc as plsc

BATCH = 16384
DIM = 128
LANES = 16
N_WORKERS = 32
ROWS_PER_W = BATCH // N_WORKERS
CHUNK = 128
N_CHUNKS = ROWS_PER_W // CHUNK
GROUPS = ROWS_PER_W // LANES
GPC = CHUNK // LANES


def _sc_body(z_hbm, j_hbm, ep_hbm, out_hbm,
             idx_all, idxp1_all, lo_all, hi_all, z_all,
             sem_z0, sem_z1, sem_z2, sem_z3,
             sem_o0, sem_o1, sem_o2, sem_o3, sem_g):
    wid = lax.axis_index("s") * 2 + lax.axis_index("c")
    base = wid * ROWS_PER_W
    sem_z = [sem_z0, sem_z1, sem_z2, sem_z3]
    sem_o = [sem_o0, sem_o1, sem_o2, sem_o3]

    fills = [
        pltpu.async_copy(
            z_hbm.at[pl.ds(base + k * CHUNK, CHUNK), :],
            z_all.at[pl.ds(k * CHUNK, CHUNK), :], sem_z[k])
        for k in range(N_CHUNKS)
    ]
    pltpu.sync_copy(j_hbm.at[pl.ds(base, ROWS_PER_W)], idx_all)
    for v in range(ROWS_PER_W // LANES):
        s = pl.ds(v * LANES, LANES)
        idxp1_all[s] = idx_all[s] + 1
    gathers = []
    for g in range(N_CHUNKS):
        s = pl.ds(g * CHUNK, CHUNK)
        gathers.append(
            pltpu.async_copy(ep_hbm.at[idx_all.at[s]], lo_all.at[s], sem_g))
        gathers.append(
            pltpu.async_copy(ep_hbm.at[idxp1_all.at[s]], hi_all.at[s], sem_g))
    for g in gathers:
        g.wait()
    fills[0].wait()

    def grp_body(gi, _):
        for k in range(1, N_CHUNKS):
            @pl.when(gi == k * GPC)
            def _():
                fills[k].wait()
        o = gi * LANES
        lo_vec = lo_all[pl.ds(o, LANES)]
        hi_vec = hi_all[pl.ds(o, LANES)]
        sc_vec = hi_vec - lo_vec
        for r in range(LANES):
            lo_s = lo_vec[r]
            sc_s = sc_vec[r]
            for v in range(DIM // LANES):
                s = pl.ds(v * LANES, LANES)
                z_all[o + r, s] = z_all[o + r, s] * sc_s + lo_s
        for k in range(N_CHUNKS):
            @pl.when(gi == (k + 1) * GPC - 1)
            def _():
                pltpu.async_copy(
                    z_all.at[pl.ds(k * CHUNK, CHUNK), :],
                    out_hbm.at[pl.ds(base + k * CHUNK, CHUNK), :], sem_o[k])
        return 0

    lax.fori_loop(0, GROUPS, grp_body, 0)
    for k in range(N_CHUNKS):
        pltpu.make_async_copy(
            z_hbm.at[pl.ds(base + k * CHUNK, CHUNK), :],
            z_all.at[pl.ds(k * CHUNK, CHUNK), :], sem_o[k]).wait()


@jax.jit
def _shift_mapper_sc(z, j_flat, endpoints):
    mesh = plsc.VectorSubcoreMesh(core_axis_name="c", subcore_axis_name="s")
    kfn = pl.kernel(
        _sc_body,
        mesh=mesh,
        out_type=jax.ShapeDtypeStruct((BATCH, DIM), jnp.float32),
        scratch_types=[
            pltpu.VMEM((ROWS_PER_W,), jnp.int32),
            pltpu.VMEM((ROWS_PER_W,), jnp.int32),
            pltpu.VMEM((ROWS_PER_W,), jnp.float32),
            pltpu.VMEM((ROWS_PER_W,), jnp.float32),
            pltpu.VMEM((ROWS_PER_W, DIM), jnp.float32),
            pltpu.SemaphoreType.DMA,
            pltpu.SemaphoreType.DMA,
            pltpu.SemaphoreType.DMA,
            pltpu.SemaphoreType.DMA,
            pltpu.SemaphoreType.DMA,
            pltpu.SemaphoreType.DMA,
            pltpu.SemaphoreType.DMA,
            pltpu.SemaphoreType.DMA,
            pltpu.SemaphoreType.DMA,
        ],
        compiler_params=pltpu.CompilerParams(needs_layout_passes=False),
    )
    return kfn(z, j_flat, endpoints)


def kernel(z, j, endpoints):
    j_flat = j.reshape(-1).astype(jnp.int32)
    return _shift_mapper_sc(z, j_flat, endpoints)


# speedup vs baseline: 1.0903x; 1.0341x over previous
import jax
import jax.numpy as jnp
from jax import lax
from jax.experimental import pallas as pl
from jax.experimental.pallas import tpu as pltpu
from jax.experimental.pallas import tpu_sc as plsc

BATCH = 16384
DIM = 128
LANES = 16
N_WORKERS = 32
ROWS_PER_W = BATCH // N_WORKERS
CHUNK = 256
N_CHUNKS = ROWS_PER_W // CHUNK
GROUPS = ROWS_PER_W // LANES
GPC = CHUNK // LANES


def _sc_body(z_hbm, j_hbm, ep_hbm, out_hbm,
             idx_all, idxp1_all, lo_all, hi_all, z_all,
             sem_z0, sem_z1, sem_z2, sem_z3,
             sem_o0, sem_o1, sem_o2, sem_o3, sem_g):
    wid = lax.axis_index("s") * 2 + lax.axis_index("c")
    base = wid * ROWS_PER_W
    sem_z = [sem_z0, sem_z1, sem_z2, sem_z3]
    sem_o = [sem_o0, sem_o1, sem_o2, sem_o3]

    fills = [
        pltpu.async_copy(
            z_hbm.at[pl.ds(base + k * CHUNK, CHUNK), :],
            z_all.at[pl.ds(k * CHUNK, CHUNK), :], sem_z[k])
        for k in range(N_CHUNKS)
    ]
    pltpu.sync_copy(j_hbm.at[pl.ds(base, ROWS_PER_W)], idx_all)
    for v in range(ROWS_PER_W // LANES):
        s = pl.ds(v * LANES, LANES)
        idxp1_all[s] = idx_all[s] + 1
    gathers = []
    for g in range(ROWS_PER_W // 128):
        s = pl.ds(g * 128, 128)
        gathers.append(
            pltpu.async_copy(ep_hbm.at[idx_all.at[s]], lo_all.at[s], sem_g))
        gathers.append(
            pltpu.async_copy(ep_hbm.at[idxp1_all.at[s]], hi_all.at[s], sem_g))
    for g in gathers:
        g.wait()
    fills[0].wait()

    def grp_body(gi, _):
        for k in range(1, N_CHUNKS):
            @pl.when(gi == k * GPC)
            def _():
                fills[k].wait()
        o = gi * LANES
        lo_vec = lo_all[pl.ds(o, LANES)]
        hi_vec = hi_all[pl.ds(o, LANES)]
        sc_vec = hi_vec - lo_vec
        for r in range(LANES):
            lo_s = lo_vec[r]
            sc_s = sc_vec[r]
            for v in range(DIM // LANES):
                s = pl.ds(v * LANES, LANES)
                z_all[o + r, s] = z_all[o + r, s] * sc_s + lo_s
        for k in range(N_CHUNKS):
            @pl.when(gi == (k + 1) * GPC - 1)
            def _():
                pltpu.async_copy(
                    z_all.at[pl.ds(k * CHUNK, CHUNK), :],
                    out_hbm.at[pl.ds(base + k * CHUNK, CHUNK), :], sem_o[k])
        return 0

    lax.fori_loop(0, GROUPS, grp_body, 0)
    for k in range(N_CHUNKS):
        pltpu.make_async_copy(
            z_hbm.at[pl.ds(base + k * CHUNK, CHUNK), :],
            z_all.at[pl.ds(k * CHUNK, CHUNK), :], sem_o[k]).wait()


@jax.jit
def _shift_mapper_sc(z, j_flat, endpoints):
    mesh = plsc.VectorSubcoreMesh(core_axis_name="c", subcore_axis_name="s")
    kfn = pl.kernel(
        _sc_body,
        mesh=mesh,
        out_type=jax.ShapeDtypeStruct((BATCH, DIM), jnp.float32),
        scratch_types=[
            pltpu.VMEM((ROWS_PER_W,), jnp.int32),
            pltpu.VMEM((ROWS_PER_W,), jnp.int32),
            pltpu.VMEM((ROWS_PER_W,), jnp.float32),
            pltpu.VMEM((ROWS_PER_W,), jnp.float32),
            pltpu.VMEM((ROWS_PER_W, DIM), jnp.float32),
            pltpu.SemaphoreType.DMA,
            pltpu.SemaphoreType.DMA,
            pltpu.SemaphoreType.DMA,
            pltpu.SemaphoreType.DMA,
            pltpu.SemaphoreType.DMA,
            pltpu.SemaphoreType.DMA,
            pltpu.SemaphoreType.DMA,
            pltpu.SemaphoreType.DMA,
            pltpu.SemaphoreType.DMA,
        ],
        compiler_params=pltpu.CompilerParams(needs_layout_passes=False),
    )
    return kfn(z, j_flat, endpoints)


def kernel(z, j, endpoints):
    j_flat = j.reshape(-1).astype(jnp.int32)
    return _shift_mapper_sc(z, j_flat, endpoints)


# speedup vs baseline: 1.1070x; 1.0153x over previous
import jax
import jax.numpy as jnp
from jax import lax
from jax.experimental import pallas as pl
from jax.experimental.pallas import tpu as pltpu
from jax.experimental.pallas import tpu_sc as plsc

BATCH = 16384
DIM = 128
LANES = 16
N_WORKERS = 32
ROWS_PER_W = BATCH // N_WORKERS
CHUNK = 256
N_CHUNKS = ROWS_PER_W // CHUNK
GROUPS = ROWS_PER_W // LANES
GPC = CHUNK // LANES
DRAIN = 128


def _sc_body(z_hbm, j_hbm, ep_hbm, out_hbm,
             idx_all, idxp1_all, lo_all, hi_all, z_all,
             sem_z0, sem_z1, sem_z2, sem_z3,
             sem_o0, sem_o1, sem_o2, sem_o3, sem_g):
    wid = lax.axis_index("s") * 2 + lax.axis_index("c")
    base = wid * ROWS_PER_W
    sem_z = [sem_z0, sem_z1, sem_z2, sem_z3]
    sem_o = [sem_o0, sem_o1, sem_o2, sem_o3]

    fills = [
        pltpu.async_copy(
            z_hbm.at[pl.ds(base + k * CHUNK, CHUNK), :],
            z_all.at[pl.ds(k * CHUNK, CHUNK), :], sem_z[k])
        for k in range(N_CHUNKS)
    ]
    pltpu.sync_copy(j_hbm.at[pl.ds(base, ROWS_PER_W)], idx_all)
    for v in range(ROWS_PER_W // LANES):
        s = pl.ds(v * LANES, LANES)
        idxp1_all[s] = idx_all[s] + 1
    gathers = []
    for g in range(ROWS_PER_W // 128):
        s = pl.ds(g * 128, 128)
        gathers.append(
            pltpu.async_copy(ep_hbm.at[idx_all.at[s]], lo_all.at[s], sem_g))
        gathers.append(
            pltpu.async_copy(ep_hbm.at[idxp1_all.at[s]], hi_all.at[s], sem_g))
    for g in gathers:
        g.wait()
    fills[0].wait()

    def grp_body(gi, _):
        for k in range(1, N_CHUNKS):
            @pl.when(gi == k * GPC)
            def _():
                fills[k].wait()
        o = gi * LANES
        lo_vec = lo_all[pl.ds(o, LANES)]
        hi_vec = hi_all[pl.ds(o, LANES)]
        sc_vec = hi_vec - lo_vec
        for r in range(LANES):
            lo_s = lo_vec[r]
            sc_s = sc_vec[r]
            for v in range(DIM // LANES):
                s = pl.ds(v * LANES, LANES)
                z_all[o + r, s] = z_all[o + r, s] * sc_s + lo_s
        for k in range(ROWS_PER_W // DRAIN):
            @pl.when(gi == (k + 1) * (DRAIN // LANES) - 1)
            def _():
                pltpu.async_copy(
                    z_all.at[pl.ds(k * DRAIN, DRAIN), :],
                    out_hbm.at[pl.ds(base + k * DRAIN, DRAIN), :], sem_o[k])
        return 0

    lax.fori_loop(0, GROUPS, grp_body, 0)
    for k in range(ROWS_PER_W // DRAIN):
        pltpu.make_async_copy(
            z_hbm.at[pl.ds(base + k * DRAIN, DRAIN), :],
            z_all.at[pl.ds(k * DRAIN, DRAIN), :], sem_o[k]).wait()


@jax.jit
def _shift_mapper_sc(z, j_flat, endpoints):
    mesh = plsc.VectorSubcoreMesh(core_axis_name="c", subcore_axis_name="s")
    kfn = pl.kernel(
        _sc_body,
        mesh=mesh,
        out_type=jax.ShapeDtypeStruct((BATCH, DIM), jnp.float32),
        scratch_types=[
            pltpu.VMEM((ROWS_PER_W,), jnp.int32),
            pltpu.VMEM((ROWS_PER_W,), jnp.int32),
            pltpu.VMEM((ROWS_PER_W,), jnp.float32),
            pltpu.VMEM((ROWS_PER_W,), jnp.float32),
            pltpu.VMEM((ROWS_PER_W, DIM), jnp.float32),
            pltpu.SemaphoreType.DMA,
            pltpu.SemaphoreType.DMA,
            pltpu.SemaphoreType.DMA,
            pltpu.SemaphoreType.DMA,
            pltpu.SemaphoreType.DMA,
            pltpu.SemaphoreType.DMA,
            pltpu.SemaphoreType.DMA,
            pltpu.SemaphoreType.DMA,
            pltpu.SemaphoreType.DMA,
        ],
        compiler_params=pltpu.CompilerParams(needs_layout_passes=False),
    )
    return kfn(z, j_flat, endpoints)


def kernel(z, j, endpoints):
    j_flat = j.reshape(-1).astype(jnp.int32)
    return _shift_mapper_sc(z, j_flat, endpoints)
